# baseline (speedup 1.0000x reference)
import functools

import jax
import jax.numpy as jnp
from jax import lax
from jax.experimental import pallas as pl
from jax.experimental.pallas import tpu as pltpu
from jax.experimental.pallas import tpu_sc as plsc

N_NODES = 10000
N_EDGES = 320000
D = 128

NC = 2
NS = 16
NW = NC * NS

CH = 125
KPW = 80

NPAD = 10240
STRIPE = NPAD // NS

_f32 = jnp.float32
_bf16 = jnp.bfloat16


def _sc_body(want_cnt, *refs):
    if want_cnt:
        (table, srcs, dsts, ones, aggp, cntp,
         src_v, dst_v, rows0, ones_v, zbuf, sh_agg, sh_cnt, sem) = refs
    else:
        (table, srcs, dsts, aggp,
         src_v, dst_v, rows0, zbuf, sh_agg, sem) = refs
    c = lax.axis_index("c")
    s = lax.axis_index("s")
    w = c * NS + s
    r0 = s * STRIPE

    @pl.loop(0, 64)
    def zfill(jj):
        for i in range(D // 16):
            zbuf[jj, pl.ds(i * 16, 16)] = jnp.zeros((16,), _f32)

    for k in range(STRIPE // 64):
        pltpu.sync_copy(zbuf, sh_agg.at[pl.ds(r0 + k * 64, 64)])
    if want_cnt:
        for k in range(STRIPE // 128):
            pltpu.sync_copy(zbuf.at[0], sh_cnt.at[pl.ds(r0 + k * 128, 128)])
    if want_cnt:
        pltpu.sync_copy(ones, ones_v)

    pltpu.sync_copy(srcs.at[pl.ds(w * KPW, KPW)], src_v)
    pltpu.sync_copy(dsts.at[pl.ds(w * KPW, KPW)], dst_v)
    plsc.subcore_barrier()

    def chunk(j, carry):
        pltpu.async_copy(table.at[src_v.at[j]], rows0, sem).wait()
        pltpu.sync_copy(rows0, sh_agg.at[dst_v.at[j]], add=True)
        if want_cnt:
            pltpu.sync_copy(ones_v, sh_cnt.at[dst_v.at[j]], add=True)
        return carry

    lax.fori_loop(0, KPW, chunk, 0)
    plsc.subcore_barrier()

    pltpu.sync_copy(sh_agg.at[pl.ds(r0, STRIPE)], aggp.at[c].at[pl.ds(r0, STRIPE)])
    if want_cnt:
        pltpu.sync_copy(sh_cnt.at[pl.ds(r0, STRIPE)], cntp.at[c].at[pl.ds(r0, STRIPE)])


def _make_sc_kernel(want_cnt):
    out_type = [jax.ShapeDtypeStruct((NC, NPAD, D), _f32)]
    if want_cnt:
        out_type.append(jax.ShapeDtypeStruct((NC, NPAD), _f32))
    scratch = [
        pltpu.VMEM((KPW, CH), jnp.int32),
        pltpu.VMEM((KPW, CH), jnp.int32),
        pltpu.VMEM((CH, D), _f32),
    ]
    if want_cnt:
        scratch.append(pltpu.VMEM((CH,), _f32))
    scratch.append(pltpu.VMEM((64, D), _f32))
    scratch.append(pltpu.VMEM_SHARED((NPAD, D), _f32))
    if want_cnt:
        scratch.append(pltpu.VMEM_SHARED((NPAD,), _f32))
    scratch.append(pltpu.SemaphoreType.DMA)
    mesh = plsc.VectorSubcoreMesh(core_axis_name="c", subcore_axis_name="s")
    return pl.kernel(
        functools.partial(_sc_body, want_cnt),
        out_type=tuple(out_type) if want_cnt else out_type[0],
        mesh=mesh,
        scratch_types=scratch,
        name="sage_sc_agg" + ("_cnt" if want_cnt else ""),
    )


_sc_agg_cnt = _make_sc_kernel(True)
_sc_agg = _make_sc_kernel(False)


def _tc_body(relu, aggp, cnt3, x, wl, bl, wr, out):
    agg = aggp[0] + aggp[1]
    cnt = cnt3[0] + cnt3[1]
    mean = agg / jnp.maximum(cnt, 1.0)
    h = (lax.dot_general(mean, wl[...], (((1,), (1,)), ((), ())),
                         preferred_element_type=_f32)
         + bl[...]
         + lax.dot_general(x[...], wr[...], (((1,), (1,)), ((), ())),
                           preferred_element_type=_f32))
    if relu:
        h = jnp.maximum(h, 0.0)
    out[...] = h


_TCR = 5000


def _tc_layer(aggp, cnt3, x, wl, bl, wr, relu):
    grid = N_NODES // _TCR
    return pl.pallas_call(
        functools.partial(_tc_body, relu),
        grid=(grid,),
        in_specs=[
            pl.BlockSpec((NC, _TCR, D), lambda i: (0, i, 0)),
            pl.BlockSpec((NC, _TCR, 1), lambda i: (0, i, 0)),
            pl.BlockSpec((_TCR, D), lambda i: (i, 0)),
            pl.BlockSpec((D, D), lambda i: (0, 0)),
            pl.BlockSpec((1, D), lambda i: (0, 0)),
            pl.BlockSpec((D, D), lambda i: (0, 0)),
        ],
        out_specs=pl.BlockSpec((_TCR, D), lambda i: (i, 0)),
        out_shape=jax.ShapeDtypeStruct((N_NODES, D), _f32),
        name="sage_tc_dense" + ("_relu" if relu else ""),
    )(aggp, cnt3, x, wl, bl, wr)


def kernel(x, edge_index, W1l, b1, W1r, W2l, b2, W2r):
    src_p = edge_index[0].astype(jnp.int32).reshape(NW * KPW, CH)
    dst_p = edge_index[1].astype(jnp.int32).reshape(NW * KPW, CH)
    ones = jnp.ones((CH,), _f32)

    aggp, cntp = _sc_agg_cnt(x, src_p, dst_p, ones)
    cnt3 = cntp.reshape(NC, NPAD, 1)
    b1r = b1.reshape(1, D)
    b2r = b2.reshape(1, D)

    h = _tc_layer(aggp, cnt3, x, W1l, b1r, W1r, relu=True)
    aggp2 = _sc_agg(h, src_p, dst_p)
    out = _tc_layer(aggp2, cnt3, h, W2l, b2r, W2r, relu=False)
    return out


# speedup vs baseline: 1.0021x; 1.0021x over previous
import functools

import jax
import jax.numpy as jnp
from jax import lax
from jax.experimental import pallas as pl
from jax.experimental.pallas import tpu as pltpu
from jax.experimental.pallas import tpu_sc as plsc

N_NODES = 10000
N_EDGES = 320000
D = 128

NC = 2
NS = 16
NW = NC * NS

CH = 125
KPW = 80

NPAD = 10240
STRIPE = NPAD // NS

_f32 = jnp.float32
_bf16 = jnp.bfloat16


def _sc_body(want_cnt, *refs):
    if want_cnt:
        (table, srcs, dsts, ones, aggp, cntp,
         src_v, dst_v, rows0, ones_v, zbuf, sh_agg, sh_cnt, sem) = refs
    else:
        (table, srcs, dsts, aggp,
         src_v, dst_v, rows0, zbuf, sh_agg, sem) = refs
    c = lax.axis_index("c")
    s = lax.axis_index("s")
    w = c * NS + s
    r0 = s * STRIPE

    @pl.loop(0, 64)
    def zfill(jj):
        for i in range(D // 16):
            zbuf[jj, pl.ds(i * 16, 16)] = jnp.zeros((16,), _f32)

    for k in range(STRIPE // 64):
        pltpu.sync_copy(zbuf, sh_agg.at[pl.ds(r0 + k * 64, 64)])
    if want_cnt:
        for k in range(STRIPE // 128):
            pltpu.sync_copy(zbuf.at[0], sh_cnt.at[pl.ds(r0 + k * 128, 128)])
    if want_cnt:
        pltpu.sync_copy(ones, ones_v)

    pltpu.sync_copy(srcs.at[pl.ds(w * KPW, KPW)], src_v)
    pltpu.sync_copy(dsts.at[pl.ds(w * KPW, KPW)], dst_v)
    plsc.subcore_barrier()

    def chunk(j, carry):
        pltpu.async_copy(table.at[src_v.at[j]], rows0, sem).wait()
        pltpu.sync_copy(rows0, sh_agg.at[dst_v.at[j]], add=True)
        if want_cnt:
            pltpu.sync_copy(ones_v, sh_cnt.at[dst_v.at[j]], add=True)
        return carry

    lax.fori_loop(0, KPW, chunk, 0)
    plsc.subcore_barrier()

    pltpu.sync_copy(sh_agg.at[pl.ds(r0, STRIPE)], aggp.at[c].at[pl.ds(r0, STRIPE)])
    if want_cnt:
        pltpu.sync_copy(sh_cnt.at[pl.ds(r0, STRIPE)], cntp.at[c].at[pl.ds(r0, STRIPE)])


def _make_sc_kernel(want_cnt):
    out_type = [jax.ShapeDtypeStruct((NC, NPAD, D), _f32)]
    if want_cnt:
        out_type.append(jax.ShapeDtypeStruct((NC, NPAD), _f32))
    scratch = [
        pltpu.VMEM((KPW, CH), jnp.int32),
        pltpu.VMEM((KPW, CH), jnp.int32),
        pltpu.VMEM((CH, D), _f32),
    ]
    if want_cnt:
        scratch.append(pltpu.VMEM((CH,), _f32))
    scratch.append(pltpu.VMEM((64, D), _f32))
    scratch.append(pltpu.VMEM_SHARED((NPAD, D), _f32))
    if want_cnt:
        scratch.append(pltpu.VMEM_SHARED((NPAD,), _f32))
    scratch.append(pltpu.SemaphoreType.DMA)
    mesh = plsc.VectorSubcoreMesh(core_axis_name="c", subcore_axis_name="s")
    return pl.kernel(
        functools.partial(_sc_body, want_cnt),
        out_type=tuple(out_type) if want_cnt else out_type[0],
        mesh=mesh,
        scratch_types=scratch,
        name="sage_sc_agg" + ("_cnt" if want_cnt else ""),
    )


_sc_agg_cnt = _make_sc_kernel(True)
_sc_agg = _make_sc_kernel(False)


def _tc_body(relu, aggp, cnt3, x, wl, bl, wr, out):
    agg = aggp[0] + aggp[1]
    cnt = cnt3[0] + cnt3[1]
    mean = agg / jnp.maximum(cnt, 1.0)
    h = (lax.dot_general(mean, wl[...], (((1,), (1,)), ((), ())),
                         preferred_element_type=_f32)
         + bl[...]
         + lax.dot_general(x[...], wr[...], (((1,), (1,)), ((), ())),
                           preferred_element_type=_f32))
    if relu:
        h = jnp.maximum(h, 0.0)
    out[...] = h


_TCR = 2000


def _tc_layer(aggp, cnt3, x, wl, bl, wr, relu):
    grid = N_NODES // _TCR
    return pl.pallas_call(
        functools.partial(_tc_body, relu),
        grid=(grid,),
        in_specs=[
            pl.BlockSpec((NC, _TCR, D), lambda i: (0, i, 0)),
            pl.BlockSpec((NC, _TCR, 1), lambda i: (0, i, 0)),
            pl.BlockSpec((_TCR, D), lambda i: (i, 0)),
            pl.BlockSpec((D, D), lambda i: (0, 0)),
            pl.BlockSpec((1, D), lambda i: (0, 0)),
            pl.BlockSpec((D, D), lambda i: (0, 0)),
        ],
        out_specs=pl.BlockSpec((_TCR, D), lambda i: (i, 0)),
        out_shape=jax.ShapeDtypeStruct((N_NODES, D), _f32),
        name="sage_tc_dense" + ("_relu" if relu else ""),
    )(aggp, cnt3, x, wl, bl, wr)


def kernel(x, edge_index, W1l, b1, W1r, W2l, b2, W2r):
    src_p = edge_index[0].astype(jnp.int32).reshape(NW * KPW, CH)
    dst_p = edge_index[1].astype(jnp.int32).reshape(NW * KPW, CH)
    ones = jnp.ones((CH,), _f32)

    aggp, cntp = _sc_agg_cnt(x, src_p, dst_p, ones)
    cnt3 = cntp.reshape(NC, NPAD, 1)
    b1r = b1.reshape(1, D)
    b2r = b2.reshape(1, D)

    h = _tc_layer(aggp, cnt3, x, W1l, b1r, W1r, relu=True)
    aggp2 = _sc_agg(h, src_p, dst_p)
    out = _tc_layer(aggp2, cnt3, h, W2l, b2r, W2r, relu=False)
    return out


# speedup vs baseline: 1.0334x; 1.0313x over previous
import functools

import jax
import jax.numpy as jnp
from jax import lax
from jax.experimental import pallas as pl
from jax.experimental.pallas import tpu as pltpu
from jax.experimental.pallas import tpu_sc as plsc

N_NODES = 10000
N_EDGES = 320000
D = 128

NC = 2
NS = 16
NW = NC * NS

CH = 125
KPW = 80

NPAD = 10240
STRIPE = NPAD // NS

_f32 = jnp.float32
_bf16 = jnp.bfloat16


def _sc_body(want_cnt, *refs):
    if want_cnt:
        (table, eidx, ones, aggp, cntp,
         src_v, dst_v, rows0, ones_v, zbuf, sh_agg, sh_cnt, sem) = refs
    else:
        (table, eidx, aggp,
         src_v, dst_v, rows0, zbuf, sh_agg, sem) = refs
    c = lax.axis_index("c")
    s = lax.axis_index("s")
    w = c * NS + s
    r0 = s * STRIPE

    @pl.loop(0, 64)
    def zfill(jj):
        for i in range(D // 16):
            zbuf[jj, pl.ds(i * 16, 16)] = jnp.zeros((16,), _f32)

    for k in range(STRIPE // 64):
        pltpu.sync_copy(zbuf, sh_agg.at[pl.ds(r0 + k * 64, 64)])
    if want_cnt:
        for k in range(STRIPE // 128):
            pltpu.sync_copy(zbuf.at[0], sh_cnt.at[pl.ds(r0 + k * 128, 128)])
    if want_cnt:
        pltpu.sync_copy(ones, ones_v)

    pltpu.sync_copy(eidx.at[0].at[pl.ds(w * KPW, KPW)], src_v)
    pltpu.sync_copy(eidx.at[1].at[pl.ds(w * KPW, KPW)], dst_v)
    plsc.subcore_barrier()

    def chunk(j, carry):
        pltpu.async_copy(table.at[src_v.at[j]], rows0, sem).wait()
        pltpu.sync_copy(rows0, sh_agg.at[dst_v.at[j]], add=True)
        if want_cnt:
            pltpu.sync_copy(ones_v, sh_cnt.at[dst_v.at[j]], add=True)
        return carry

    lax.fori_loop(0, KPW, chunk, 0)
    plsc.subcore_barrier()

    pltpu.sync_copy(sh_agg.at[pl.ds(r0, STRIPE)], aggp.at[c].at[pl.ds(r0, STRIPE)])
    if want_cnt:
        pltpu.sync_copy(sh_cnt.at[pl.ds(r0, STRIPE)], cntp.at[c].at[pl.ds(r0, STRIPE)])


def _make_sc_kernel(want_cnt):
    out_type = [jax.ShapeDtypeStruct((NC, NPAD, D), _f32)]
    if want_cnt:
        out_type.append(jax.ShapeDtypeStruct((NC, NPAD), _f32))
    scratch = [
        pltpu.VMEM((KPW, CH), jnp.int32),
        pltpu.VMEM((KPW, CH), jnp.int32),
        pltpu.VMEM((CH, D), _f32),
    ]
    if want_cnt:
        scratch.append(pltpu.VMEM((CH,), _f32))
    scratch.append(pltpu.VMEM((64, D), _f32))
    scratch.append(pltpu.VMEM_SHARED((NPAD, D), _f32))
    if want_cnt:
        scratch.append(pltpu.VMEM_SHARED((NPAD,), _f32))
    scratch.append(pltpu.SemaphoreType.DMA)
    mesh = plsc.VectorSubcoreMesh(core_axis_name="c", subcore_axis_name="s")
    return pl.kernel(
        functools.partial(_sc_body, want_cnt),
        out_type=tuple(out_type) if want_cnt else out_type[0],
        mesh=mesh,
        scratch_types=scratch,
        name="sage_sc_agg" + ("_cnt" if want_cnt else ""),
    )


_sc_agg_cnt = _make_sc_kernel(True)
_sc_agg = _make_sc_kernel(False)


def _tc_body(relu, aggp, cnt3, x, wl, bl, wr, out):
    agg = aggp[0] + aggp[1]
    cnt = cnt3[0] + cnt3[1]
    mean = agg / jnp.maximum(cnt, 1.0)
    h = (lax.dot_general(mean, wl[...], (((1,), (1,)), ((), ())),
                         preferred_element_type=_f32)
         + bl[...]
         + lax.dot_general(x[...], wr[...], (((1,), (1,)), ((), ())),
                           preferred_element_type=_f32))
    if relu:
        h = jnp.maximum(h, 0.0)
    out[...] = h


_TCR = 2000


def _tc_layer(aggp, cnt3, x, wl, bl, wr, relu):
    grid = N_NODES // _TCR
    return pl.pallas_call(
        functools.partial(_tc_body, relu),
        grid=(grid,),
        in_specs=[
            pl.BlockSpec((NC, _TCR, D), lambda i: (0, i, 0)),
            pl.BlockSpec((NC, _TCR, 1), lambda i: (0, i, 0)),
            pl.BlockSpec((_TCR, D), lambda i: (i, 0)),
            pl.BlockSpec((D, D), lambda i: (0, 0)),
            pl.BlockSpec((1, D), lambda i: (0, 0)),
            pl.BlockSpec((D, D), lambda i: (0, 0)),
        ],
        out_specs=pl.BlockSpec((_TCR, D), lambda i: (i, 0)),
        out_shape=jax.ShapeDtypeStruct((N_NODES, D), _f32),
        name="sage_tc_dense" + ("_relu" if relu else ""),
    )(aggp, cnt3, x, wl, bl, wr)


def kernel(x, edge_index, W1l, b1, W1r, W2l, b2, W2r):
    eidx = edge_index.astype(jnp.int32).reshape(2, NW * KPW, CH)
    ones = jnp.ones((CH,), _f32)

    aggp, cntp = _sc_agg_cnt(x, eidx, ones)
    cnt3 = cntp.reshape(NC, NPAD, 1)
    b1r = b1.reshape(1, D)
    b2r = b2.reshape(1, D)

    h = _tc_layer(aggp, cnt3, x, W1l, b1r, W1r, relu=True)
    aggp2 = _sc_agg(h, eidx)
    out = _tc_layer(aggp2, cnt3, h, W2l, b2r, W2r, relu=False)
    return out
